# Initial kernel scaffold; baseline (speedup 1.0000x reference)
import jax
import jax.numpy as jnp
from jax.experimental import pallas as pl


def kernel(x, edge_index, batch, lin0_w, lin0_b, conv_ws, lin1_w, lin1_b):
    raise NotImplementedError("write your pallas kernel here")


# speedup vs baseline: 1.0193x; 1.0193x over previous
import functools
import math

import jax
import jax.numpy as jnp
from jax.experimental import pallas as pl
from jax.experimental.pallas import tpu as pltpu

N = 10000
E = 160000
IN_C = 256
HID = 512
OUT_C = 64
NUM_LAYERS = 8
ALPHA = 0.5
THETA = 1.0
NUM_GRAPHS = 128

ROW_BLK = 1000
GRID = N // ROW_BLK


def _lin0_body(x_ref, w_ref, b_ref, o_ref):
    o_ref[...] = jnp.maximum(
        jnp.dot(x_ref[...], w_ref[...], preferred_element_type=jnp.float32)
        + b_ref[...], 0.0)


def _lin0(x, w, b):
    return pl.pallas_call(
        _lin0_body,
        grid=(GRID,),
        in_specs=[
            pl.BlockSpec((ROW_BLK, IN_C), lambda i: (i, 0)),
            pl.BlockSpec((IN_C, HID), lambda i: (0, 0)),
            pl.BlockSpec((1, HID), lambda i: (0, 0)),
        ],
        out_specs=pl.BlockSpec((ROW_BLK, HID), lambda i: (i, 0)),
        out_shape=jax.ShapeDtypeStruct((N, HID), jnp.float32),
    )(x, w, b.reshape(1, HID))


def _layer_body(beta, agg_ref, x0_ref, h_ref, w_ref, o_ref):
    out = agg_ref[...] * (1.0 - ALPHA) + ALPHA * x0_ref[...]
    y = (1.0 - beta) * out + beta * jnp.dot(
        out, w_ref[...], preferred_element_type=jnp.float32)
    o_ref[...] = jnp.maximum(y + h_ref[...], 0.0)


def _layer(agg, x0, h, w, beta):
    return pl.pallas_call(
        functools.partial(_layer_body, beta),
        grid=(GRID,),
        in_specs=[
            pl.BlockSpec((ROW_BLK, HID), lambda i: (i, 0)),
            pl.BlockSpec((ROW_BLK, HID), lambda i: (i, 0)),
            pl.BlockSpec((ROW_BLK, HID), lambda i: (i, 0)),
            pl.BlockSpec((HID, HID), lambda i: (0, 0)),
        ],
        out_specs=pl.BlockSpec((ROW_BLK, HID), lambda i: (i, 0)),
        out_shape=jax.ShapeDtypeStruct((N, HID), jnp.float32),
    )(agg, x0, h, w)


def _pool_head_body(h_ref, batch_ref, w_ref, b_ref, o_ref, sums, counts):
    i = pl.program_id(0)

    @pl.when(i == 0)
    def _init():
        sums[...] = jnp.zeros_like(sums)
        counts[...] = jnp.zeros_like(counts)

    seg = batch_ref[0]
    gids = jax.lax.broadcasted_iota(jnp.int32, (NUM_GRAPHS, ROW_BLK), 0)
    onehot = (gids == seg).astype(jnp.float32)
    sums[...] += jnp.dot(onehot, h_ref[...], preferred_element_type=jnp.float32)
    counts[...] += jnp.sum(onehot, axis=1, keepdims=True)

    @pl.when(i == GRID - 1)
    def _fin():
        pooled = sums[...] / jnp.clip(counts[...], 1.0, None)
        logits = jnp.dot(pooled, w_ref[...],
                         preferred_element_type=jnp.float32) + b_ref[...]
        m = jnp.max(logits, axis=-1, keepdims=True)
        z = logits - m
        lse = jnp.log(jnp.sum(jnp.exp(z), axis=-1, keepdims=True))
        o_ref[...] = z - lse


def _pool_head(h, batch, w, b):
    return pl.pallas_call(
        _pool_head_body,
        grid=(GRID,),
        in_specs=[
            pl.BlockSpec((ROW_BLK, HID), lambda i: (i, 0)),
            pl.BlockSpec((1, 1, ROW_BLK), lambda i: (i, 0, 0)),
            pl.BlockSpec((HID, OUT_C), lambda i: (0, 0)),
            pl.BlockSpec((1, OUT_C), lambda i: (0, 0)),
        ],
        out_specs=pl.BlockSpec((NUM_GRAPHS, OUT_C), lambda i: (0, 0)),
        out_shape=jax.ShapeDtypeStruct((NUM_GRAPHS, OUT_C), jnp.float32),
        scratch_shapes=[
            pltpu.VMEM((NUM_GRAPHS, HID), jnp.float32),
            pltpu.VMEM((NUM_GRAPHS, 1), jnp.float32),
        ],
    )(h, batch.reshape(GRID, 1, ROW_BLK), w, b.reshape(1, OUT_C))


def kernel(x, edge_index, batch, lin0_w, lin0_b, conv_ws, lin1_w, lin1_b):
    src = edge_index[0]
    dst = edge_index[1]
    h = _lin0(x, lin0_w, lin0_b)
    x0 = h
    for layer in range(NUM_LAYERS):
        beta = float(math.log(THETA / (layer + 1) + 1.0))
        agg = jax.ops.segment_sum(h[src], dst, num_segments=N)
        h = _layer(agg, x0, h, conv_ws[layer], beta)
    return _pool_head(h, batch, lin1_w, lin1_b)


# speedup vs baseline: 3.1421x; 3.0824x over previous
import functools
import math

import jax
import jax.numpy as jnp
from jax import lax
from jax.experimental import pallas as pl
from jax.experimental.pallas import tpu as pltpu
from jax.experimental.pallas import tpu_sc as plsc

N = 10000
E = 160000
IN_C = 256
HID = 512
OUT_C = 64
NUM_LAYERS = 8
ALPHA = 0.5
THETA = 1.0
NUM_GRAPHS = 128

FC = 128
NCHUNK = HID // FC
NTILE = 16
BATCH = 128
EPT = 10112
NBATCH = EPT // BATCH
E_PAD = EPT * NTILE
AGG_ROWS = NTILE * 632

ROW_BLK = 1000
GRID = N // ROW_BLK


def _spmm_body(h0, h1, h2, h3, srcb, dstb, zeros, a0, a1, a2, a3,
               src_v, dst_v, rows_v, gsem, agg):
    cid = lax.axis_index("c")
    sid = lax.axis_index("s")
    pltpu.sync_copy(srcb.at[sid], src_v)
    pltpu.sync_copy(dstb.at[sid], dst_v)
    h_refs = (h0, h1, h2, h3)
    o_refs = (a0, a1, a2, a3)
    for chunk in range(NCHUNK):
        @pl.when(cid == chunk // 2)
        def _(h_ref=h_refs[chunk], o_ref=o_refs[chunk]):
            pltpu.sync_copy(zeros.at[pl.ds(sid * 632, 632)],
                            agg.at[pl.ds(sid * 632, 632)])
            plsc.subcore_barrier()

            def batch_body(j, carry):
                pltpu.async_copy(h_ref.at[src_v.at[j]], rows_v, gsem).wait()
                pltpu.sync_copy(rows_v, agg.at[dst_v.at[j]], add=True)
                return carry

            lax.fori_loop(0, NBATCH, batch_body, 0)
            plsc.subcore_barrier()

            @pl.when(sid < NTILE - 1)
            def _copy_main():
                pltpu.sync_copy(agg.at[pl.ds(sid * 624, 624)],
                                o_ref.at[pl.ds(sid * 624, 624)])

            @pl.when(sid == NTILE - 1)
            def _copy_last():
                pltpu.sync_copy(agg.at[pl.ds(9360, 640)],
                                o_ref.at[pl.ds(9360, 640)])

            plsc.subcore_barrier()


_spmm_call = pl.kernel(
    _spmm_body,
    out_type=tuple(jax.ShapeDtypeStruct((N, FC), jnp.float32)
                   for _ in range(NCHUNK)),
    mesh=plsc.VectorSubcoreMesh(core_axis_name="c", subcore_axis_name="s"),
    scratch_types=[
        pltpu.VMEM((NBATCH, BATCH), jnp.int32),
        pltpu.VMEM((NBATCH, BATCH), jnp.int32),
        pltpu.VMEM((BATCH, FC), jnp.float32),
        pltpu.SemaphoreType.DMA,
        pltpu.VMEM_SHARED((AGG_ROWS, FC), jnp.float32),
    ],
)


def _split_store(o_refs, y):
    for j, o in enumerate(o_refs):
        o[...] = y[:, j * FC:(j + 1) * FC]


def _cat(refs):
    return jnp.concatenate([r[...] for r in refs], axis=1)


def _lin0_body(x_ref, w_ref, b_ref, *o_refs):
    y = jnp.maximum(
        jnp.dot(x_ref[...], w_ref[...], preferred_element_type=jnp.float32)
        + b_ref[...], 0.0)
    _split_store(o_refs, y)


def _lin0(x, w, b):
    return pl.pallas_call(
        _lin0_body,
        grid=(GRID,),
        in_specs=[
            pl.BlockSpec((ROW_BLK, IN_C), lambda i: (i, 0)),
            pl.BlockSpec((IN_C, HID), lambda i: (0, 0)),
            pl.BlockSpec((1, HID), lambda i: (0, 0)),
        ],
        out_specs=[pl.BlockSpec((ROW_BLK, FC), lambda i: (i, 0))] * NCHUNK,
        out_shape=[jax.ShapeDtypeStruct((N, FC), jnp.float32)] * NCHUNK,
    )(x, w, b.reshape(1, HID))


def _layer_body(beta, *refs):
    agg_refs = refs[0:4]
    x0_refs = refs[4:8]
    h_refs = refs[8:12]
    w_ref = refs[12]
    o_refs = refs[13:17]
    out = _cat(agg_refs) * (1.0 - ALPHA) + ALPHA * _cat(x0_refs)
    y = (1.0 - beta) * out + beta * jnp.dot(
        out, w_ref[...], preferred_element_type=jnp.float32)
    _split_store(o_refs, jnp.maximum(y + _cat(h_refs), 0.0))


def _layer(aggs, x0s, hs, w, beta):
    blk = pl.BlockSpec((ROW_BLK, FC), lambda i: (i, 0))
    return pl.pallas_call(
        functools.partial(_layer_body, beta),
        grid=(GRID,),
        in_specs=[blk] * 12 + [pl.BlockSpec((HID, HID), lambda i: (0, 0))],
        out_specs=[blk] * NCHUNK,
        out_shape=[jax.ShapeDtypeStruct((N, FC), jnp.float32)] * NCHUNK,
    )(*aggs, *x0s, *hs, w)


def _pool_head_body(h0, h1, h2, h3, batch_ref, w_ref, b_ref, o_ref,
                    sums, counts):
    i = pl.program_id(0)

    @pl.when(i == 0)
    def _init():
        sums[...] = jnp.zeros_like(sums)
        counts[...] = jnp.zeros_like(counts)

    seg = batch_ref[0]
    gids = lax.broadcasted_iota(jnp.int32, (NUM_GRAPHS, ROW_BLK), 0)
    onehot = (gids == seg).astype(jnp.float32)
    h = _cat((h0, h1, h2, h3))
    sums[...] += jnp.dot(onehot, h, preferred_element_type=jnp.float32)
    counts[...] += jnp.sum(onehot, axis=1, keepdims=True)

    @pl.when(i == GRID - 1)
    def _fin():
        pooled = sums[...] / jnp.clip(counts[...], 1.0, None)
        logits = jnp.dot(pooled, w_ref[...],
                         preferred_element_type=jnp.float32) + b_ref[...]
        m = jnp.max(logits, axis=-1, keepdims=True)
        z = logits - m
        lse = jnp.log(jnp.sum(jnp.exp(z), axis=-1, keepdims=True))
        o_ref[...] = z - lse


def _pool_head(hs, batch, w, b):
    blk = pl.BlockSpec((ROW_BLK, FC), lambda i: (i, 0))
    return pl.pallas_call(
        _pool_head_body,
        grid=(GRID,),
        in_specs=[blk] * NCHUNK + [
            pl.BlockSpec((1, 1, ROW_BLK), lambda i: (i, 0, 0)),
            pl.BlockSpec((HID, OUT_C), lambda i: (0, 0)),
            pl.BlockSpec((1, OUT_C), lambda i: (0, 0)),
        ],
        out_specs=pl.BlockSpec((NUM_GRAPHS, OUT_C), lambda i: (0, 0)),
        out_shape=jax.ShapeDtypeStruct((NUM_GRAPHS, OUT_C), jnp.float32),
        scratch_shapes=[
            pltpu.VMEM((NUM_GRAPHS, HID), jnp.float32),
            pltpu.VMEM((NUM_GRAPHS, 1), jnp.float32),
        ],
    )(*hs, batch.reshape(GRID, 1, ROW_BLK), w, b.reshape(1, OUT_C))


def kernel(x, edge_index, batch, lin0_w, lin0_b, conv_ws, lin1_w, lin1_b):
    src = edge_index[0]
    dst = edge_index[1]
    npad = E_PAD - E
    srcb = jnp.concatenate(
        [src, jnp.zeros((npad,), jnp.int32)]).reshape(NTILE, NBATCH, BATCH)
    dstb = jnp.concatenate(
        [dst, jnp.full((npad,), N, jnp.int32)]).reshape(NTILE, NBATCH, BATCH)
    zeros = jnp.zeros((AGG_ROWS, FC), jnp.float32)

    hs = _lin0(x, lin0_w, lin0_b)
    x0s = hs
    for layer in range(NUM_LAYERS):
        beta = float(math.log(THETA / (layer + 1) + 1.0))
        aggs = _spmm_call(*hs, srcb, dstb, zeros)
        hs = _layer(aggs, x0s, hs, conv_ws[layer], beta)
    return _pool_head(hs, batch, lin1_w, lin1_b)
